# baseline (speedup 1.0000x reference)
import dataclasses
import functools

import jax
import jax.numpy as jnp
from jax import lax
from jax.experimental import pallas as pl
from jax.experimental.pallas import tpu as pltpu
from jax.experimental.pallas import tpu_sc as plsc

N = 10000
D = 128
E = 320000

NC = 2
NS = 16
LANES = 16
NW = NC * NS

K = 128
NBUF = 2
NCHUNK0 = 56
NCHUNK1 = 104
MAXC = max(NCHUNK0, NCHUNK1)
EPW = (NCHUNK0 + NCHUNK1) * K // 2
EP = NW * EPW

N_PAD = 10240
ROWS_PER_TILE = N_PAD // NS
ZCHUNK = K

_MESH = plsc.VectorSubcoreMesh(
    core_axis_name="c", subcore_axis_name="s", num_cores=NC, num_subcores=NS)

_SC_PARAMS = pltpu.CompilerParams()
if "needs_layout_passes" in pltpu.CompilerParams.__dataclass_fields__:
    _SC_PARAMS = dataclasses.replace(_SC_PARAMS, needs_layout_passes=False)


def _gemm_body(h_ref, w_ref, norm_ref, g_ref):
    g_ref[...] = jnp.dot(
        h_ref[...], w_ref[...], preferred_element_type=jnp.float32
    ) * norm_ref[...]


def _compute_g(h, W, norm):
    M = 1000
    return pl.pallas_call(
        _gemm_body,
        grid=(N // M,),
        in_specs=[
            pl.BlockSpec((M, D), lambda i: (i, i * 0)),
            pl.BlockSpec((D, D), lambda i: (i * 0, i * 0)),
            pl.BlockSpec((M, 1), lambda i: (i, i * 0)),
        ],
        out_specs=pl.BlockSpec((M, D), lambda i: (i, i * 0)),
        out_shape=jax.ShapeDtypeStruct((N, D), jnp.float32),
    )(h, W, norm)


NCHUNK = EPW // K


def _floop(n, body, unroll=None):
    def wrapped(i, carry):
        body(i)
        return carry
    lax.fori_loop(jnp.int32(0), jnp.int32(n), wrapped, None, unroll=unroll)


def _floopd(n, body):
    def wrapped(i, carry):
        body(i)
        return carry
    lax.fori_loop(jnp.int32(0), n, wrapped, None)


def _sc_body(g_hbm, src_hbm, dst_hbm, w_hbm, out_hbm,
             src_all, dstb, wb, rows, acc, gsems, dsems, wsems):
    i32 = jnp.int32
    cid = lax.axis_index("c").astype(i32)
    sid = lax.axis_index("s").astype(i32)
    wid = sid * i32(NC) + cid

    zero16 = jnp.zeros((LANES,), jnp.float32)

    def _zero_row(r):
        for c in range(D // LANES):
            rows[0][r, pl.ds(c * LANES, LANES)] = zero16
    _floop(K, _zero_row)

    tile_base = sid * i32(ROWS_PER_TILE)
    for j in range(ROWS_PER_TILE // ZCHUNK):
        pltpu.sync_copy(
            rows[0].at[pl.ds(0, ZCHUNK)],
            acc.at[pl.ds(tile_base + i32(j * ZCHUNK), ZCHUNK)],
        )

    pltpu.sync_copy(src_hbm.at[wid], src_all)

    plsc.subcore_barrier()

    def _idx_prefetch(i, p):
        pltpu.async_copy(dst_hbm.at[wid, i], dstb.at[i32(p)], dsems[p])
        pltpu.async_copy(w_hbm.at[wid, i], wb.at[pl.ds(p * K, K)], wsems[p])

    def _idx_wait(p):
        z = i32(0)
        pltpu.make_async_copy(
            dst_hbm.at[wid, z], dstb.at[i32(p)], dsems[p]).wait()
        pltpu.make_async_copy(
            w_hbm.at[wid, z], wb.at[pl.ds(p * K, K)], wsems[p]).wait()

    def _gather_start(i, p):
        pltpu.async_copy(g_hbm.at[src_all.at[i]], rows[p], gsems[p])

    def _gather_wait(p):
        pltpu.make_async_copy(
            g_hbm.at[src_all.at[jnp.int32(0)]], rows[p], gsems[p]).wait()

    def _scale(p):
        base = i32(p * K)
        U = 4
        def body(eu):
            e0 = eu * i32(U)
            for u in range(U):
                e = e0 + i32(u)
                eidx = jnp.broadcast_to(base + e, (LANES,))
                wv = plsc.load_gather(wb, [eidx])
                for c in range(D // LANES):
                    sl = pl.ds(c * LANES, LANES)
                    rows[p][e, sl] = rows[p][e, sl] * wv
        _floop(K // U, body)

    ncka = jnp.where(cid == i32(0), i32(NCHUNK0), i32(NCHUNK1))
    for p in range(NBUF):
        _idx_prefetch(i32(p), p)
    for p in range(NBUF - 1):
        _gather_start(i32(p), p)

    def _quad(j):
        i0 = j * i32(NBUF)
        for p in range(NBUF):
            i = i0 + i32(p)
            _gather_wait(p)
            _idx_wait(p)
            _scale(p)
            pltpu.sync_copy(rows[p], acc.at[dstb.at[i32(p)]], add=True)
            _idx_prefetch(i + i32(NBUF), p)
            _gather_start(i + i32(NBUF - 1), (p + NBUF - 1) % NBUF)

    _floopd(ncka // i32(NBUF), _quad)
    for p in range(NBUF - 1):
        _gather_wait(p)
    for p in range(NBUF):
        _idx_wait(p)

    plsc.subcore_barrier()

    for j in range(ROWS_PER_TILE // ZCHUNK):
        sl = pl.ds(tile_base + i32(j * ZCHUNK), ZCHUNK)
        pltpu.sync_copy(acc.at[sl], out_hbm.at[cid, sl])


@functools.partial(
    pl.kernel,
    out_type=jax.ShapeDtypeStruct((NC, N_PAD, D), jnp.float32),
    mesh=_MESH,
    scratch_types=[
        pltpu.VMEM((MAXC + NBUF, K), jnp.int32),
        pltpu.VMEM((NBUF, K), jnp.int32),
        pltpu.VMEM((NBUF * K,), jnp.float32),
        [pltpu.VMEM((K, D), jnp.float32)] * NBUF,
        pltpu.VMEM_SHARED((N_PAD, D), jnp.float32),
        [pltpu.SemaphoreType.DMA] * NBUF,
        [pltpu.SemaphoreType.DMA] * NBUF,
        [pltpu.SemaphoreType.DMA] * NBUF,
    ],
    compiler_params=_SC_PARAMS,
)
def _sc_scatter(g_hbm, src_hbm, dst_hbm, w_hbm, out_hbm,
                src_all, dstb, wb, rows, acc, gsems, dsems, wsems):
    _sc_body(g_hbm, src_hbm, dst_hbm, w_hbm, out_hbm,
             src_all, dstb, wb, rows, acc, gsems, dsems, wsems)


def _combine_body(p_ref, norm_ref, b_ref, o_ref):
    o_ref[...] = (p_ref[0] + p_ref[1]) * norm_ref[...] + b_ref[...]


def _combine(partials, norm, b2d):
    M = 1000
    return pl.pallas_call(
        _combine_body,
        grid=(N // M,),
        in_specs=[
            pl.BlockSpec((NC, M, D), lambda i: (i * 0, i, i * 0)),
            pl.BlockSpec((M, 1), lambda i: (i, i * 0)),
            pl.BlockSpec((1, D), lambda i: (i * 0, i * 0)),
        ],
        out_specs=pl.BlockSpec((M, D), lambda i: (i, i * 0)),
        out_shape=jax.ShapeDtypeStruct((N, D), jnp.float32),
    )(partials, norm, b2d)


def kernel(h, edge_index, norm, edge_weight, W, b):
    src = edge_index[0].astype(jnp.int32)
    dst = edge_index[1].astype(jnp.int32)
    w = edge_weight.reshape(E).astype(jnp.float32)

    pad = EP - E
    src = jnp.concatenate([src, jnp.zeros((pad,), jnp.int32)])
    dst = jnp.concatenate([dst, jnp.zeros((pad,), jnp.int32)])
    w = jnp.concatenate([w, jnp.zeros((pad,), jnp.float32)])

    def _pack(a, dtype):
        a0 = a[:NS * NCHUNK0 * K].reshape(NS, NCHUNK0, K)
        a1 = a[NS * NCHUNK0 * K:].reshape(NS, NCHUNK1, K)
        z0 = jnp.zeros((NS, MAXC + NBUF - NCHUNK0, K), dtype)
        z1 = jnp.zeros((NS, MAXC + NBUF - NCHUNK1, K), dtype)
        a0 = jnp.concatenate([a0, z0], axis=1)
        a1 = jnp.concatenate([a1, z1], axis=1)
        return jnp.stack([a0, a1], axis=1).reshape(NW, MAXC + NBUF, K)

    src = _pack(src, jnp.int32)
    dst = _pack(dst, jnp.int32)
    w = _pack(w, jnp.float32)

    g = _compute_g(h, W, norm)
    partials = _sc_scatter(g, src, dst, w)
    return _combine(partials, norm, b.reshape(1, D))


# speedup vs baseline: 1.0802x; 1.0802x over previous
import dataclasses
import functools

import jax
import jax.numpy as jnp
from jax import lax
from jax.experimental import pallas as pl
from jax.experimental.pallas import tpu as pltpu
from jax.experimental.pallas import tpu_sc as plsc

N = 10000
D = 128
E = 320000

NC = 2
NS = 16
LANES = 16
NW = NC * NS

K = 128
NBUF = 2
NCHUNK0 = 80
NCHUNK1 = 80
MAXC = max(NCHUNK0, NCHUNK1)
EPW = (NCHUNK0 + NCHUNK1) * K // 2
EP = NW * EPW

N_PAD = 10240
ROWS_PER_TILE = N_PAD // NS
ZCHUNK = K

_MESH = plsc.VectorSubcoreMesh(
    core_axis_name="c", subcore_axis_name="s", num_cores=NC, num_subcores=NS)

_SC_PARAMS = pltpu.CompilerParams()
if "needs_layout_passes" in pltpu.CompilerParams.__dataclass_fields__:
    _SC_PARAMS = dataclasses.replace(_SC_PARAMS, needs_layout_passes=False)


def _gemm_body(h_ref, w_ref, norm_ref, g_ref):
    g_ref[...] = jnp.dot(
        h_ref[...], w_ref[...], preferred_element_type=jnp.float32
    ) * norm_ref[...]


def _compute_g(h, W, norm):
    M = 1000
    return pl.pallas_call(
        _gemm_body,
        grid=(N // M,),
        in_specs=[
            pl.BlockSpec((M, D), lambda i: (i, i * 0)),
            pl.BlockSpec((D, D), lambda i: (i * 0, i * 0)),
            pl.BlockSpec((M, 1), lambda i: (i, i * 0)),
        ],
        out_specs=pl.BlockSpec((M, D), lambda i: (i, i * 0)),
        out_shape=jax.ShapeDtypeStruct((N, D), jnp.float32),
    )(h, W, norm)


NCHUNK = EPW // K


def _floop(n, body, unroll=None):
    def wrapped(i, carry):
        body(i)
        return carry
    lax.fori_loop(jnp.int32(0), jnp.int32(n), wrapped, None, unroll=unroll)


def _floopd(n, body):
    def wrapped(i, carry):
        body(i)
        return carry
    lax.fori_loop(jnp.int32(0), n, wrapped, None)


def _sc_body(g_hbm, src_hbm, dst_hbm, w_hbm, out_hbm,
             src_all, dstb, wb, rows, acc, gsems, dsems, wsems):
    i32 = jnp.int32
    cid = lax.axis_index("c").astype(i32)
    sid = lax.axis_index("s").astype(i32)
    wid = sid * i32(NC) + cid

    zero16 = jnp.zeros((LANES,), jnp.float32)

    def _zero_row(r):
        for c in range(D // LANES):
            rows[0][r, pl.ds(c * LANES, LANES)] = zero16
    _floop(K, _zero_row)

    tile_base = sid * i32(ROWS_PER_TILE)
    for j in range(ROWS_PER_TILE // ZCHUNK):
        pltpu.sync_copy(
            rows[0].at[pl.ds(0, ZCHUNK)],
            acc.at[pl.ds(tile_base + i32(j * ZCHUNK), ZCHUNK)],
        )

    pltpu.sync_copy(src_hbm.at[wid], src_all)

    plsc.subcore_barrier()

    def _idx_prefetch(i, p):
        pltpu.async_copy(dst_hbm.at[wid, i], dstb.at[i32(p)], dsems[p])
        pltpu.async_copy(w_hbm.at[wid, i], wb.at[pl.ds(p * K, K)], wsems[p])

    def _idx_wait(p):
        z = i32(0)
        pltpu.make_async_copy(
            dst_hbm.at[wid, z], dstb.at[i32(p)], dsems[p]).wait()
        pltpu.make_async_copy(
            w_hbm.at[wid, z], wb.at[pl.ds(p * K, K)], wsems[p]).wait()

    def _gather_start(i, p):
        pltpu.async_copy(g_hbm.at[src_all.at[i]], rows[p], gsems[p])

    def _gather_wait(p):
        pltpu.make_async_copy(
            g_hbm.at[src_all.at[jnp.int32(0)]], rows[p], gsems[p]).wait()

    def _scale(p):
        base = i32(p * K)
        U = 4
        def body(eu):
            e0 = eu * i32(U)
            for u in range(U):
                e = e0 + i32(u)
                eidx = jnp.broadcast_to(base + e, (LANES,))
                wv = plsc.load_gather(wb, [eidx])
                for c in range(D // LANES):
                    sl = pl.ds(c * LANES, LANES)
                    rows[p][e, sl] = rows[p][e, sl] * wv
        _floop(K // U, body)

    for p in range(NBUF):
        _idx_prefetch(i32(p), p)
    for p in range(NBUF - 1):
        _gather_start(i32(p), p)

    def _quad(j):
        i0 = j * i32(NBUF)
        for p in range(NBUF):
            i = i0 + i32(p)
            _gather_wait(p)
            _idx_wait(p)
            _scale(p)
            pltpu.sync_copy(rows[p], acc.at[dstb.at[i32(p)]], add=True)
            _idx_prefetch(i + i32(NBUF), p)
            _gather_start(i + i32(NBUF - 1), (p + NBUF - 1) % NBUF)

    _floop(MAXC // NBUF, _quad)
    for p in range(NBUF - 1):
        _gather_wait(p)
    for p in range(NBUF):
        _idx_wait(p)

    plsc.subcore_barrier()

    for j in range(ROWS_PER_TILE // ZCHUNK):
        sl = pl.ds(tile_base + i32(j * ZCHUNK), ZCHUNK)
        pltpu.sync_copy(acc.at[sl], out_hbm.at[cid, sl])


@functools.partial(
    pl.kernel,
    out_type=jax.ShapeDtypeStruct((NC, N_PAD, D), jnp.float32),
    mesh=_MESH,
    scratch_types=[
        pltpu.VMEM((MAXC + NBUF, K), jnp.int32),
        pltpu.VMEM((NBUF, K), jnp.int32),
        pltpu.VMEM((NBUF * K,), jnp.float32),
        [pltpu.VMEM((K, D), jnp.float32)] * NBUF,
        pltpu.VMEM_SHARED((N_PAD, D), jnp.float32),
        [pltpu.SemaphoreType.DMA] * NBUF,
        [pltpu.SemaphoreType.DMA] * NBUF,
        [pltpu.SemaphoreType.DMA] * NBUF,
    ],
    compiler_params=_SC_PARAMS,
)
def _sc_scatter(g_hbm, src_hbm, dst_hbm, w_hbm, out_hbm,
                src_all, dstb, wb, rows, acc, gsems, dsems, wsems):
    _sc_body(g_hbm, src_hbm, dst_hbm, w_hbm, out_hbm,
             src_all, dstb, wb, rows, acc, gsems, dsems, wsems)


def _combine_body(p_ref, norm_ref, b_ref, o_ref):
    o_ref[...] = (p_ref[0] + p_ref[1]) * norm_ref[...] + b_ref[...]


def _combine(partials, norm, b2d):
    M = 1000
    return pl.pallas_call(
        _combine_body,
        grid=(N // M,),
        in_specs=[
            pl.BlockSpec((NC, M, D), lambda i: (i * 0, i, i * 0)),
            pl.BlockSpec((M, 1), lambda i: (i, i * 0)),
            pl.BlockSpec((1, D), lambda i: (i * 0, i * 0)),
        ],
        out_specs=pl.BlockSpec((M, D), lambda i: (i, i * 0)),
        out_shape=jax.ShapeDtypeStruct((N, D), jnp.float32),
    )(partials, norm, b2d)


def kernel(h, edge_index, norm, edge_weight, W, b):
    src = edge_index[0].astype(jnp.int32)
    dst = edge_index[1].astype(jnp.int32)
    w = edge_weight.reshape(E).astype(jnp.float32)

    pad = EP - E
    src = jnp.concatenate([src, jnp.zeros((pad,), jnp.int32)])
    dst = jnp.concatenate([dst, jnp.zeros((pad,), jnp.int32)])
    w = jnp.concatenate([w, jnp.zeros((pad,), jnp.float32)])

    def _pack(a, dtype):
        a0 = a[:NS * NCHUNK0 * K].reshape(NS, NCHUNK0, K)
        a1 = a[NS * NCHUNK0 * K:].reshape(NS, NCHUNK1, K)
        z0 = jnp.zeros((NS, MAXC + NBUF - NCHUNK0, K), dtype)
        z1 = jnp.zeros((NS, MAXC + NBUF - NCHUNK1, K), dtype)
        a0 = jnp.concatenate([a0, z0], axis=1)
        a1 = jnp.concatenate([a1, z1], axis=1)
        return jnp.stack([a0, a1], axis=1).reshape(NW, MAXC + NBUF, K)

    src = _pack(src, jnp.int32)
    dst = _pack(dst, jnp.int32)
    w = _pack(w, jnp.float32)

    g = _compute_g(h, W, norm)
    partials = _sc_scatter(g, src, dst, w)
    return _combine(partials, norm, b.reshape(1, D))


# speedup vs baseline: 1.2920x; 1.1960x over previous
import dataclasses
import functools

import jax
import jax.numpy as jnp
from jax import lax
from jax.experimental import pallas as pl
from jax.experimental.pallas import tpu as pltpu
from jax.experimental.pallas import tpu_sc as plsc

N = 10000
D = 128
E = 320000

NC = 2
NS = 16
LANES = 16
NW = NC * NS

K = 128
NBUF = 2
NCHUNK0 = 80
NCHUNK1 = 80
MAXC = max(NCHUNK0, NCHUNK1)
EPW = (NCHUNK0 + NCHUNK1) * K // 2
EP = NW * EPW

N_PAD = 10240
ROWS_PER_TILE = N_PAD // NS
ZCHUNK = K

_MESH = plsc.VectorSubcoreMesh(
    core_axis_name="c", subcore_axis_name="s", num_cores=NC, num_subcores=NS)

_SC_PARAMS = pltpu.CompilerParams()
if "needs_layout_passes" in pltpu.CompilerParams.__dataclass_fields__:
    _SC_PARAMS = dataclasses.replace(_SC_PARAMS, needs_layout_passes=False)


def _gemm_body(h_ref, w_ref, norm_ref, g_ref):
    g_ref[...] = jnp.dot(
        h_ref[...], w_ref[...], preferred_element_type=jnp.float32
    ) * norm_ref[...]


def _compute_g(h, W, norm):
    M = 1000
    return pl.pallas_call(
        _gemm_body,
        grid=(N // M,),
        in_specs=[
            pl.BlockSpec((M, D), lambda i: (i, i * 0)),
            pl.BlockSpec((D, D), lambda i: (i * 0, i * 0)),
            pl.BlockSpec((M, 1), lambda i: (i, i * 0)),
        ],
        out_specs=pl.BlockSpec((M, D), lambda i: (i, i * 0)),
        out_shape=jax.ShapeDtypeStruct((N, D), jnp.float32),
    )(h, W, norm)


NCHUNK = EPW // K


def _floop(n, body, unroll=None):
    def wrapped(i, carry):
        body(i)
        return carry
    lax.fori_loop(jnp.int32(0), jnp.int32(n), wrapped, None, unroll=unroll)


def _floopd(n, body):
    def wrapped(i, carry):
        body(i)
        return carry
    lax.fori_loop(jnp.int32(0), n, wrapped, None)


def _sc_body(g_hbm, src_hbm, dst_hbm, w_hbm, out_hbm,
             src_all, dstb, wb, rows, acc, gsems, dsems, wsems):
    i32 = jnp.int32
    cid = lax.axis_index("c").astype(i32)
    sid = lax.axis_index("s").astype(i32)
    wid = sid * i32(NC) + cid

    zero16 = jnp.zeros((LANES,), jnp.float32)

    def _zero_row(r):
        for c in range(D // LANES):
            rows[0][r, pl.ds(c * LANES, LANES)] = zero16
    _floop(K, _zero_row)

    tile_base = sid * i32(ROWS_PER_TILE)
    for j in range(ROWS_PER_TILE // ZCHUNK):
        pltpu.sync_copy(
            rows[0].at[pl.ds(0, ZCHUNK)],
            acc.at[pl.ds(tile_base + i32(j * ZCHUNK), ZCHUNK)],
        )

    pltpu.sync_copy(src_hbm.at[wid], src_all)

    plsc.subcore_barrier()

    def _idx_prefetch(i, p):
        pltpu.async_copy(dst_hbm.at[wid, i], dstb.at[i32(p)], dsems[p])
        pltpu.async_copy(w_hbm.at[wid, i], wb.at[pl.ds(p * K, K)], wsems[p])

    def _idx_wait(p):
        z = i32(0)
        pltpu.make_async_copy(
            dst_hbm.at[wid, z], dstb.at[i32(p)], dsems[p]).wait()
        pltpu.make_async_copy(
            w_hbm.at[wid, z], wb.at[pl.ds(p * K, K)], wsems[p]).wait()

    def _gather_start(i, p):
        pltpu.async_copy(g_hbm.at[src_all.at[i]], rows[p], gsems[p])

    def _gather_wait(p):
        pltpu.make_async_copy(
            g_hbm.at[src_all.at[jnp.int32(0)]], rows[p], gsems[p]).wait()

    def _scale(p):
        base = i32(p * K)
        U = 4
        def body(eu):
            e0 = eu * i32(U)
            for u in range(U):
                e = e0 + i32(u)
                eidx = jnp.broadcast_to(base + e, (LANES,))
                wv = plsc.load_gather(wb, [eidx])
                for c in range(D // LANES):
                    sl = pl.ds(c * LANES, LANES)
                    rows[p][e, sl] = rows[p][e, sl] * wv
        _floop(K // U, body)

    for p in range(NBUF):
        _idx_prefetch(i32(p), p)
    for p in range(NBUF - 1):
        _gather_start(i32(p), p)

    def _quad(j):
        i0 = j * i32(NBUF)
        for p in range(NBUF):
            i = i0 + i32(p)
            _gather_start(i + i32(NBUF - 1), (p + NBUF - 1) % NBUF)
            _gather_wait(p)
            _idx_wait(p)
            _scale(p)
            pltpu.sync_copy(rows[p], acc.at[dstb.at[i32(p)]], add=True)
            _idx_prefetch(i + i32(NBUF), p)

    _floop(MAXC // NBUF, _quad)
    for p in range(NBUF - 1):
        _gather_wait(p)
    for p in range(NBUF):
        _idx_wait(p)

    plsc.subcore_barrier()

    for j in range(ROWS_PER_TILE // ZCHUNK):
        sl = pl.ds(tile_base + i32(j * ZCHUNK), ZCHUNK)
        pltpu.sync_copy(acc.at[sl], out_hbm.at[cid, sl])


@functools.partial(
    pl.kernel,
    out_type=jax.ShapeDtypeStruct((NC, N_PAD, D), jnp.float32),
    mesh=_MESH,
    scratch_types=[
        pltpu.VMEM((MAXC + NBUF, K), jnp.int32),
        pltpu.VMEM((NBUF, K), jnp.int32),
        pltpu.VMEM((NBUF * K,), jnp.float32),
        [pltpu.VMEM((K, D), jnp.float32)] * NBUF,
        pltpu.VMEM_SHARED((N_PAD, D), jnp.float32),
        [pltpu.SemaphoreType.DMA] * NBUF,
        [pltpu.SemaphoreType.DMA] * NBUF,
        [pltpu.SemaphoreType.DMA] * NBUF,
    ],
    compiler_params=_SC_PARAMS,
)
def _sc_scatter(g_hbm, src_hbm, dst_hbm, w_hbm, out_hbm,
                src_all, dstb, wb, rows, acc, gsems, dsems, wsems):
    _sc_body(g_hbm, src_hbm, dst_hbm, w_hbm, out_hbm,
             src_all, dstb, wb, rows, acc, gsems, dsems, wsems)


def _combine_body(p_ref, norm_ref, b_ref, o_ref):
    o_ref[...] = (p_ref[0] + p_ref[1]) * norm_ref[...] + b_ref[...]


def _combine(partials, norm, b2d):
    M = 1000
    return pl.pallas_call(
        _combine_body,
        grid=(N // M,),
        in_specs=[
            pl.BlockSpec((NC, M, D), lambda i: (i * 0, i, i * 0)),
            pl.BlockSpec((M, 1), lambda i: (i, i * 0)),
            pl.BlockSpec((1, D), lambda i: (i * 0, i * 0)),
        ],
        out_specs=pl.BlockSpec((M, D), lambda i: (i, i * 0)),
        out_shape=jax.ShapeDtypeStruct((N, D), jnp.float32),
    )(partials, norm, b2d)


def kernel(h, edge_index, norm, edge_weight, W, b):
    src = edge_index[0].astype(jnp.int32)
    dst = edge_index[1].astype(jnp.int32)
    w = edge_weight.reshape(E).astype(jnp.float32)

    pad = EP - E
    src = jnp.concatenate([src, jnp.zeros((pad,), jnp.int32)])
    dst = jnp.concatenate([dst, jnp.zeros((pad,), jnp.int32)])
    w = jnp.concatenate([w, jnp.zeros((pad,), jnp.float32)])

    def _pack(a, dtype):
        a0 = a[:NS * NCHUNK0 * K].reshape(NS, NCHUNK0, K)
        a1 = a[NS * NCHUNK0 * K:].reshape(NS, NCHUNK1, K)
        z0 = jnp.zeros((NS, MAXC + NBUF - NCHUNK0, K), dtype)
        z1 = jnp.zeros((NS, MAXC + NBUF - NCHUNK1, K), dtype)
        a0 = jnp.concatenate([a0, z0], axis=1)
        a1 = jnp.concatenate([a1, z1], axis=1)
        return jnp.stack([a0, a1], axis=1).reshape(NW, MAXC + NBUF, K)

    src = _pack(src, jnp.int32)
    dst = _pack(dst, jnp.int32)
    w = _pack(w, jnp.float32)

    g = _compute_g(h, W, norm)
    partials = _sc_scatter(g, src, dst, w)
    return _combine(partials, norm, b.reshape(1, D))
